# baseline (speedup 1.0000x reference)
import functools

import jax
import jax.numpy as jnp
from jax import lax
from jax.experimental import pallas as pl
from jax.experimental.pallas import tpu as pltpu
from jax.experimental.pallas import tpu_sc as plsc

D = H = W = 128
N = D * H * W
NC, NS = 2, 16
NW = NC * NS
PER_W = N // NW
CHUNK = 8192
NCHUNK = PER_W // CHUNK
LANES = 16

TC_COLS = 128
TC_ROWS = N // TC_COLS
TC_BLOCK = 4096
TC_GRID = TC_ROWS // TC_BLOCK


def _remap(s, lanes):
    five = jnp.full(lanes, 5, jnp.int32)
    zero = jnp.zeros(lanes, jnp.int32)
    return jnp.where(s < 5, s, jnp.where(s == 18, five, zero))


def _sc_kernel(slab_hbm, lab_hbm, so_hbm, lo_hbm,
               slab_v, lab_v, so_v, lo_v,
               sem_i0, sem_i1, sem_o0, sem_o1):
    wid = lax.axis_index("s") * NC + lax.axis_index("c")
    sem_in = (sem_i0, sem_i1)
    sem_out = (sem_o0, sem_o1)

    def start_in(ci, slot):
        base = wid * PER_W + ci * CHUNK
        sl = pl.ds(base, CHUNK)
        pltpu.async_copy(slab_hbm.at[sl], slab_v.at[slot], sem_in[slot])
        pltpu.async_copy(lab_hbm.at[sl], lab_v.at[slot], sem_in[slot])

    def wait_in(slot):
        sl = pl.ds(0, CHUNK)
        pltpu.make_async_copy(slab_hbm.at[sl], slab_v.at[slot], sem_in[slot]).wait()
        pltpu.make_async_copy(lab_hbm.at[sl], lab_v.at[slot], sem_in[slot]).wait()

    def start_out(ci, slot):
        base = wid * PER_W + ci * CHUNK
        sl = pl.ds(base, CHUNK)
        pltpu.async_copy(so_v.at[slot], so_hbm.at[sl], sem_out[slot])
        pltpu.async_copy(lo_v.at[slot], lo_hbm.at[sl], sem_out[slot])

    def wait_out(slot):
        sl = pl.ds(0, CHUNK)
        pltpu.make_async_copy(so_v.at[slot], so_hbm.at[sl], sem_out[slot]).wait()
        pltpu.make_async_copy(lo_v.at[slot], lo_hbm.at[sl], sem_out[slot]).wait()

    def compute(slot):
        @plsc.parallel_loop(0, CHUNK // LANES, unroll=4)
        def _(i):
            off = i * LANES
            s = slab_v[slot, pl.ds(off, LANES)]
            so_v[slot, pl.ds(off, LANES)] = _remap(s, (LANES,))
            lo_v[slot, pl.ds(off, LANES)] = _remap(
                lab_v[slot, pl.ds(off, LANES)], (LANES,))

    start_in(0, 0)
    start_in(1, 1)

    def chunk_pair(i, _):
        ci0 = i * 2
        for b in (0, 1):
            ci = ci0 + b
            wait_in(b)

            @pl.when(ci >= 2)
            def _():
                wait_out(b)

            compute(b)
            start_out(ci, b)

            @pl.when(ci + 2 < NCHUNK)
            def _():
                start_in(ci + 2, b)

        return 0

    lax.fori_loop(0, NCHUNK // 2, chunk_pair, 0)
    wait_out(0)
    wait_out(1)


@jax.jit
def _run_sc(slab_f, lab_f):
    mesh = plsc.VectorSubcoreMesh(core_axis_name="c", subcore_axis_name="s")
    k = functools.partial(
        pl.kernel, mesh=mesh,
        compiler_params=pltpu.CompilerParams(needs_layout_passes=False),
        out_type=(
            jax.ShapeDtypeStruct((N,), jnp.int32),
            jax.ShapeDtypeStruct((N,), jnp.int32),
        ),
        scratch_types=[
            pltpu.VMEM((2, CHUNK), jnp.int32),
            pltpu.VMEM((2, CHUNK), jnp.int32),
            pltpu.VMEM((2, CHUNK), jnp.int32),
            pltpu.VMEM((2, CHUNK), jnp.int32),
            pltpu.SemaphoreType.DMA,
            pltpu.SemaphoreType.DMA,
            pltpu.SemaphoreType.DMA,
            pltpu.SemaphoreType.DMA,
        ],
    )(_sc_kernel)
    return k(slab_f, lab_f)


def _tc_kernel(tab_ref, slab_ref, noise_ref, img_ref, roi_ref,
               simg_ref, img_o_ref, roi_o_ref):
    s = slab_ref[...]
    w = jnp.zeros_like(s)
    for c in range(19):
        w = jnp.where(s == c, tab_ref[c], w)
    mu_v = lax.bitcast_convert_type(w & jnp.int32(-65536), jnp.float32)
    sg_v = lax.bitcast_convert_type(w << 16, jnp.float32)
    simg_ref[...] = mu_v + sg_v * noise_ref[...]
    img_o_ref[...] = img_ref[...]
    roi_o_ref[...] = roi_ref[...]


@jax.jit
def _run_tc(tab, slab_f, noise_f, img_f, roi_f):
    slab2 = slab_f.reshape(TC_ROWS, TC_COLS)
    noise2 = noise_f.reshape(TC_ROWS, TC_COLS)
    img2 = img_f.reshape(TC_ROWS, TC_COLS)
    roi2 = roi_f.reshape(TC_ROWS, TC_COLS)
    bs = pl.BlockSpec((TC_BLOCK, TC_COLS), lambda i: (i, 0))
    simg2, img_o2, roi_o2 = pl.pallas_call(
        _tc_kernel,
        grid=(TC_GRID,),
        in_specs=[
            pl.BlockSpec(memory_space=pltpu.SMEM),
            bs, bs, bs, bs,
        ],
        out_specs=(bs, bs, bs),
        out_shape=(
            jax.ShapeDtypeStruct((TC_ROWS, TC_COLS), jnp.float32),
            jax.ShapeDtypeStruct((TC_ROWS, TC_COLS), jnp.float32),
            jax.ShapeDtypeStruct((TC_ROWS, TC_COLS), jnp.int32),
        ),
    )(tab, slab2, noise2, img2, roi2)
    return simg2, img_o2, roi_o2


def kernel(slab, img, lab, roi, mu, sigma, noise):
    slab_f = slab.reshape(N).astype(jnp.int32)
    lab_f = lab.reshape(N).astype(jnp.int32)
    noise_f = noise.reshape(N).astype(jnp.float32)
    img_f = img.reshape(N).astype(jnp.float32)
    roi_f = roi.reshape(N).astype(jnp.int32)
    mu_bits = lax.bitcast_convert_type(
        mu.astype(jnp.float32).astype(jnp.bfloat16), jnp.uint16
    ).astype(jnp.int32)
    sg_bits = lax.bitcast_convert_type(
        sigma.astype(jnp.float32).astype(jnp.bfloat16), jnp.uint16
    ).astype(jnp.int32)
    tab = jnp.pad((mu_bits << 16) | sg_bits, (0, 32 - mu.shape[0]))

    so_f, lo_f = _run_sc(slab_f, lab_f)
    simg2, img_o2, roi_o2 = _run_tc(tab, slab_f, noise_f, img_f, roi_f)

    simg = simg2.reshape(1, D, H, W)
    img_out = img_o2.reshape(1, D, H, W)
    roi_out = roi_o2.reshape(1, D, H, W).astype(roi.dtype)
    slab_out = so_f.reshape(1, D, H, W).astype(slab.dtype)
    rlab_out = lo_f.reshape(1, D, H, W).astype(lab.dtype)
    return (simg, slab_out, img_out, rlab_out, roi_out)


# speedup vs baseline: 1.0498x; 1.0498x over previous
import functools

import jax
import jax.numpy as jnp
from jax import lax
from jax.experimental import pallas as pl
from jax.experimental.pallas import tpu as pltpu
from jax.experimental.pallas import tpu_sc as plsc

D = H = W = 128
N = D * H * W
NC, NS = 2, 16
NW = NC * NS
PER_W = N // NW
CHUNK = 8192
NCHUNK = PER_W // CHUNK
LANES = 16

TC_COLS = 128
TC_ROWS = N // TC_COLS
TC_BLOCK = 2048
TC_GRID = TC_ROWS // TC_BLOCK


def _remap(s, lanes):
    five = jnp.full(lanes, 5, jnp.int32)
    zero = jnp.zeros(lanes, jnp.int32)
    return jnp.where(s < 5, s, jnp.where(s == 18, five, zero))


def _sc_kernel(lab_hbm, lo_hbm, lab_v, lo_v, sem_i0, sem_i1, sem_o0, sem_o1):
    wid = lax.axis_index("s") * NC + lax.axis_index("c")
    sem_in = (sem_i0, sem_i1)
    sem_out = (sem_o0, sem_o1)

    def start_in(ci, slot):
        base = wid * PER_W + ci * CHUNK
        pltpu.async_copy(lab_hbm.at[pl.ds(base, CHUNK)], lab_v.at[slot],
                         sem_in[slot])

    def wait_in(slot):
        pltpu.make_async_copy(lab_hbm.at[pl.ds(0, CHUNK)], lab_v.at[slot],
                              sem_in[slot]).wait()

    def start_out(ci, slot):
        base = wid * PER_W + ci * CHUNK
        pltpu.async_copy(lo_v.at[slot], lo_hbm.at[pl.ds(base, CHUNK)],
                         sem_out[slot])

    def wait_out(slot):
        pltpu.make_async_copy(lo_v.at[slot], lo_hbm.at[pl.ds(0, CHUNK)],
                              sem_out[slot]).wait()

    def compute(slot):
        @plsc.parallel_loop(0, CHUNK // LANES, unroll=4)
        def _(i):
            off = i * LANES
            lo_v[slot, pl.ds(off, LANES)] = _remap(
                lab_v[slot, pl.ds(off, LANES)], (LANES,))

    start_in(0, 0)
    start_in(1, 1)

    def chunk_pair(i, _):
        ci0 = i * 2
        for b in (0, 1):
            ci = ci0 + b
            wait_in(b)

            @pl.when(ci >= 2)
            def _():
                wait_out(b)

            compute(b)
            start_out(ci, b)

            @pl.when(ci + 2 < NCHUNK)
            def _():
                start_in(ci + 2, b)

        return 0

    lax.fori_loop(0, NCHUNK // 2, chunk_pair, 0)
    wait_out(0)
    wait_out(1)


@jax.jit
def _run_sc(lab_f):
    mesh = plsc.VectorSubcoreMesh(core_axis_name="c", subcore_axis_name="s")
    k = functools.partial(
        pl.kernel, mesh=mesh,
        compiler_params=pltpu.CompilerParams(needs_layout_passes=False),
        out_type=jax.ShapeDtypeStruct((N,), jnp.int32),
        scratch_types=[
            pltpu.VMEM((2, CHUNK), jnp.int32),
            pltpu.VMEM((2, CHUNK), jnp.int32),
            pltpu.SemaphoreType.DMA,
            pltpu.SemaphoreType.DMA,
            pltpu.SemaphoreType.DMA,
            pltpu.SemaphoreType.DMA,
        ],
    )(_sc_kernel)
    return k(lab_f)


def _tc_kernel(tab_ref, slab_ref, noise_ref, img_ref, roi_ref,
               simg_ref, so_ref, img_o_ref, roi_o_ref):
    s = slab_ref[...]
    w = jnp.zeros_like(s)
    for c in range(19):
        w = jnp.where(s == c, tab_ref[c], w)
    mu_v = lax.bitcast_convert_type(w & jnp.int32(-65536), jnp.float32)
    sg_v = lax.bitcast_convert_type(w << 16, jnp.float32)
    simg_ref[...] = mu_v + sg_v * noise_ref[...]
    so_ref[...] = _remap(s, s.shape)
    img_o_ref[...] = img_ref[...]
    roi_o_ref[...] = roi_ref[...]


@jax.jit
def _run_tc(tab, slab_f, noise_f, img_f, roi_f):
    slab2 = slab_f.reshape(TC_ROWS, TC_COLS)
    noise2 = noise_f.reshape(TC_ROWS, TC_COLS)
    img2 = img_f.reshape(TC_ROWS, TC_COLS)
    roi2 = roi_f.reshape(TC_ROWS, TC_COLS)
    bs = pl.BlockSpec((TC_BLOCK, TC_COLS), lambda i: (i, 0))
    simg2, so2, img_o2, roi_o2 = pl.pallas_call(
        _tc_kernel,
        grid=(TC_GRID,),
        in_specs=[
            pl.BlockSpec(memory_space=pltpu.SMEM),
            bs, bs, bs, bs,
        ],
        out_specs=(bs, bs, bs, bs),
        out_shape=(
            jax.ShapeDtypeStruct((TC_ROWS, TC_COLS), jnp.float32),
            jax.ShapeDtypeStruct((TC_ROWS, TC_COLS), jnp.int32),
            jax.ShapeDtypeStruct((TC_ROWS, TC_COLS), jnp.float32),
            jax.ShapeDtypeStruct((TC_ROWS, TC_COLS), jnp.int32),
        ),
    )(tab, slab2, noise2, img2, roi2)
    return simg2, so2, img_o2, roi_o2


def kernel(slab, img, lab, roi, mu, sigma, noise):
    slab_f = slab.reshape(N).astype(jnp.int32)
    lab_f = lab.reshape(N).astype(jnp.int32)
    noise_f = noise.reshape(N).astype(jnp.float32)
    img_f = img.reshape(N).astype(jnp.float32)
    roi_f = roi.reshape(N).astype(jnp.int32)
    mu_bits = lax.bitcast_convert_type(
        mu.astype(jnp.float32).astype(jnp.bfloat16), jnp.uint16
    ).astype(jnp.int32)
    sg_bits = lax.bitcast_convert_type(
        sigma.astype(jnp.float32).astype(jnp.bfloat16), jnp.uint16
    ).astype(jnp.int32)
    tab = jnp.pad((mu_bits << 16) | sg_bits, (0, 32 - mu.shape[0]))

    lo_f = _run_sc(lab_f)
    simg2, so2, img_o2, roi_o2 = _run_tc(tab, slab_f, noise_f, img_f, roi_f)

    simg = simg2.reshape(1, D, H, W)
    slab_out = so2.reshape(1, D, H, W).astype(slab.dtype)
    img_out = img_o2.reshape(1, D, H, W)
    roi_out = roi_o2.reshape(1, D, H, W).astype(roi.dtype)
    rlab_out = lo_f.reshape(1, D, H, W).astype(lab.dtype)
    return (simg, slab_out, img_out, rlab_out, roi_out)


# speedup vs baseline: 1.0641x; 1.0136x over previous
import functools

import jax
import jax.numpy as jnp
from jax import lax
from jax.experimental import pallas as pl
from jax.experimental.pallas import tpu as pltpu
from jax.experimental.pallas import tpu_sc as plsc

D = H = W = 128
N = D * H * W
NC, NS = 2, 16
NW = NC * NS
PER_W = N // NW
CHUNK = 8192
NCHUNK = PER_W // CHUNK
LANES = 16

TC_COLS = 128
TC_ROWS = N // TC_COLS
TC_BLOCK = 2048
TC_GRID = TC_ROWS // TC_BLOCK


def _remap(s, lanes):
    five = jnp.full(lanes, 5, jnp.int32)
    zero = jnp.zeros(lanes, jnp.int32)
    return jnp.where(s < 5, s, jnp.where(s == 18, five, zero))


def _sc_kernel(lab_hbm, roi_hbm, lo_hbm, roi_o_hbm,
               lab_v, lo_v, roi_v, roi_o_v, sem_i0, sem_i1, sem_o0, sem_o1):
    wid = lax.axis_index("s") * NC + lax.axis_index("c")
    sem_in = (sem_i0, sem_i1)
    sem_out = (sem_o0, sem_o1)

    def start_in(ci, slot):
        base = wid * PER_W + ci * CHUNK
        sl = pl.ds(base, CHUNK)
        pltpu.async_copy(lab_hbm.at[sl], lab_v.at[slot], sem_in[slot])
        pltpu.async_copy(roi_hbm.at[sl], roi_v.at[slot], sem_in[slot])

    def wait_in(slot):
        sl = pl.ds(0, CHUNK)
        pltpu.make_async_copy(lab_hbm.at[sl], lab_v.at[slot],
                              sem_in[slot]).wait()
        pltpu.make_async_copy(roi_hbm.at[sl], roi_v.at[slot],
                              sem_in[slot]).wait()

    def start_out(ci, slot):
        base = wid * PER_W + ci * CHUNK
        sl = pl.ds(base, CHUNK)
        pltpu.async_copy(lo_v.at[slot], lo_hbm.at[sl], sem_out[slot])
        pltpu.async_copy(roi_o_v.at[slot], roi_o_hbm.at[sl], sem_out[slot])

    def wait_out(slot):
        sl = pl.ds(0, CHUNK)
        pltpu.make_async_copy(lo_v.at[slot], lo_hbm.at[sl],
                              sem_out[slot]).wait()
        pltpu.make_async_copy(roi_o_v.at[slot], roi_o_hbm.at[sl],
                              sem_out[slot]).wait()

    def compute(slot):
        @plsc.parallel_loop(0, CHUNK // LANES, unroll=4)
        def _(i):
            off = i * LANES
            lo_v[slot, pl.ds(off, LANES)] = _remap(
                lab_v[slot, pl.ds(off, LANES)], (LANES,))
            roi_o_v[slot, pl.ds(off, LANES)] = roi_v[slot, pl.ds(off, LANES)]

    start_in(0, 0)
    start_in(1, 1)

    def chunk_pair(i, _):
        ci0 = i * 2
        for b in (0, 1):
            ci = ci0 + b
            wait_in(b)

            @pl.when(ci >= 2)
            def _():
                wait_out(b)

            compute(b)
            start_out(ci, b)

            @pl.when(ci + 2 < NCHUNK)
            def _():
                start_in(ci + 2, b)

        return 0

    lax.fori_loop(0, NCHUNK // 2, chunk_pair, 0)
    wait_out(0)
    wait_out(1)


@jax.jit
def _run_sc(lab_f, roi_f):
    mesh = plsc.VectorSubcoreMesh(core_axis_name="c", subcore_axis_name="s")
    k = functools.partial(
        pl.kernel, mesh=mesh,
        compiler_params=pltpu.CompilerParams(needs_layout_passes=False),
        out_type=(
            jax.ShapeDtypeStruct((N,), jnp.int32),
            jax.ShapeDtypeStruct((N,), jnp.int32),
        ),
        scratch_types=[
            pltpu.VMEM((2, CHUNK), jnp.int32),
            pltpu.VMEM((2, CHUNK), jnp.int32),
            pltpu.VMEM((2, CHUNK), jnp.int32),
            pltpu.VMEM((2, CHUNK), jnp.int32),
            pltpu.SemaphoreType.DMA,
            pltpu.SemaphoreType.DMA,
            pltpu.SemaphoreType.DMA,
            pltpu.SemaphoreType.DMA,
        ],
    )(_sc_kernel)
    return k(lab_f, roi_f)


def _tc_kernel(tab_ref, slab_ref, noise_ref, img_ref,
               simg_ref, so_ref, img_o_ref):
    s = slab_ref[...]
    w = jnp.zeros_like(s)
    for c in range(19):
        w = jnp.where(s == c, tab_ref[c], w)
    mu_v = lax.bitcast_convert_type(w & jnp.int32(-65536), jnp.float32)
    sg_v = lax.bitcast_convert_type(w << 16, jnp.float32)
    simg_ref[...] = mu_v + sg_v * noise_ref[...]
    so_ref[...] = _remap(s, s.shape)
    img_o_ref[...] = img_ref[...]


@jax.jit
def _run_tc(tab, slab_f, noise_f, img_f):
    slab2 = slab_f.reshape(TC_ROWS, TC_COLS)
    noise2 = noise_f.reshape(TC_ROWS, TC_COLS)
    img2 = img_f.reshape(TC_ROWS, TC_COLS)
    bs = pl.BlockSpec((TC_BLOCK, TC_COLS), lambda i: (i, 0))
    simg2, so2, img_o2 = pl.pallas_call(
        _tc_kernel,
        grid=(TC_GRID,),
        in_specs=[
            pl.BlockSpec(memory_space=pltpu.SMEM),
            bs, bs, bs,
        ],
        out_specs=(bs, bs, bs),
        out_shape=(
            jax.ShapeDtypeStruct((TC_ROWS, TC_COLS), jnp.float32),
            jax.ShapeDtypeStruct((TC_ROWS, TC_COLS), jnp.int32),
            jax.ShapeDtypeStruct((TC_ROWS, TC_COLS), jnp.float32),
        ),
    )(tab, slab2, noise2, img2)
    return simg2, so2, img_o2


def kernel(slab, img, lab, roi, mu, sigma, noise):
    slab_f = slab.reshape(N).astype(jnp.int32)
    lab_f = lab.reshape(N).astype(jnp.int32)
    noise_f = noise.reshape(N).astype(jnp.float32)
    img_f = img.reshape(N).astype(jnp.float32)
    roi_f = roi.reshape(N).astype(jnp.int32)
    mu_bits = lax.bitcast_convert_type(
        mu.astype(jnp.float32).astype(jnp.bfloat16), jnp.uint16
    ).astype(jnp.int32)
    sg_bits = lax.bitcast_convert_type(
        sigma.astype(jnp.float32).astype(jnp.bfloat16), jnp.uint16
    ).astype(jnp.int32)
    tab = jnp.pad((mu_bits << 16) | sg_bits, (0, 32 - mu.shape[0]))

    lo_f, roi_of = _run_sc(lab_f, roi_f)
    simg2, so2, img_o2 = _run_tc(tab, slab_f, noise_f, img_f)

    simg = simg2.reshape(1, D, H, W)
    slab_out = so2.reshape(1, D, H, W).astype(slab.dtype)
    img_out = img_o2.reshape(1, D, H, W)
    roi_out = roi_of.reshape(1, D, H, W).astype(roi.dtype)
    rlab_out = lo_f.reshape(1, D, H, W).astype(lab.dtype)
    return (simg, slab_out, img_out, rlab_out, roi_out)
